# Initial kernel scaffold; baseline (speedup 1.0000x reference)
import jax
import jax.numpy as jnp
from jax.experimental import pallas as pl


def kernel(h0, h1, h2, h3_minus, h3_plus, h4, a010_row, a010_col, a010_val, a101_row, a101_col, a101_val, a232_row, a232_col, a232_val, b01_row, b01_col, b01_val, b02_row, b02_col, b02_val, b03_row, b03_col, b03_val, b04_row, b04_col, b04_val, b12_row, b12_col, b12_val, b13_row, b13_col, b13_val, b14_row, b14_col, b14_val, b23_row, b23_col, b23_val, b24_row, b24_col, b24_val, m2to0, m2to1, m2to4, params):
    raise NotImplementedError("write your pallas kernel here")


# speedup vs baseline: 1.0109x; 1.0109x over previous
import functools

import jax
import jax.numpy as jnp
from jax.experimental import pallas as pl

C = 128
_N0, _N1, _N2, _N3, _N4 = 20000, 40000, 10000, 5000, 2500
_MATS = {"a010": (_N0, _N0), "a101": (_N1, _N1), "a232": (_N2, _N2),
         "b01": (_N0, _N1), "b02": (_N0, _N2), "b03": (_N0, _N3),
         "b04": (_N0, _N4), "b12": (_N1, _N2), "b13": (_N1, _N3),
         "b14": (_N1, _N4), "b23": (_N2, _N3), "b24": (_N2, _N4)}


def _mlp2_body(x_ref, w1_ref, b1_ref, w2_ref, b2_ref, o_ref):
    x = x_ref[...]
    y = jnp.dot(x, w1_ref[...], preferred_element_type=jnp.float32)
    y = jnp.maximum(y + b1_ref[...], 0.0)
    z = jnp.dot(y, w2_ref[...], preferred_element_type=jnp.float32)
    o_ref[...] = z + b2_ref[...]


@functools.partial(jax.jit, static_argnames=("bm",))
def _mlp2(x, w1, b1, w2, b2, bm=512):
    n, din = x.shape
    dout = w2.shape[1]
    grid = (pl.cdiv(n, bm),)
    return pl.pallas_call(
        _mlp2_body,
        grid=grid,
        in_specs=[
            pl.BlockSpec((bm, din), lambda i: (i, 0)),
            pl.BlockSpec((din, C), lambda i: (0, 0)),
            pl.BlockSpec((1, C), lambda i: (0, 0)),
            pl.BlockSpec((C, dout), lambda i: (0, 0)),
            pl.BlockSpec((1, dout), lambda i: (0, 0)),
        ],
        out_specs=pl.BlockSpec((bm, dout), lambda i: (i, 0)),
        out_shape=jax.ShapeDtypeStruct((n, dout), jnp.float32),
    )(x, w1, b1, w2, b2)


def _apply_mlp(p, x):
    (w1, b1), (w2, b2) = p
    return _mlp2(x, w1, b1[None, :], w2, b2[None, :])


def _propagate(msgs, src, dst, vals, n_out, mean=False):
    m = jnp.take(msgs, src, axis=0)
    m = vals[:, None] * m
    out = jax.ops.segment_sum(m, dst, num_segments=n_out)
    if mean:
        cnt = jax.ops.segment_sum(jnp.ones_like(vals), dst, num_segments=n_out)
        out = out / jnp.clip(cnt, 1.0, None)[:, None]
    return out


def kernel(h0, h1, h2, h3_minus, h3_plus, h4, a010_row, a010_col, a010_val, a101_row, a101_col, a101_val, a232_row, a232_col, a232_val, b01_row, b01_col, b01_val, b02_row, b02_col, b02_val, b03_row, b03_col, b03_val, b04_row, b04_col, b04_val, b12_row, b12_col, b12_val, b13_row, b13_col, b13_val, b14_row, b14_col, b14_val, b23_row, b23_col, b23_val, b24_row, b24_col, b24_val, m2to0, m2to1, m2to4, params):
    P = params
    idx = {"a010": (a010_row, a010_col, a010_val),
           "a101": (a101_row, a101_col, a101_val),
           "a232": (a232_row, a232_col, a232_val),
           "b01": (b01_row, b01_col, b01_val),
           "b02": (b02_row, b02_col, b02_val),
           "b03": (b03_row, b03_col, b03_val),
           "b04": (b04_row, b04_col, b04_val),
           "b12": (b12_row, b12_col, b12_val),
           "b13": (b13_row, b13_col, b13_val),
           "b14": (b14_row, b14_col, b14_val),
           "b23": (b23_row, b23_col, b23_val),
           "b24": (b24_row, b24_col, b24_val)}

    def pr(name, msgs, T=False, mean=False):
        row, col, val = idx[name]
        src, dst = row, col
        nr, nc = _MATS[name]
        n_out = nc
        if T:
            src, dst, n_out = dst, src, nr
        return _propagate(msgs, src, dst, val, n_out, mean)

    t0to0 = _apply_mlp(P["p_0to0"], h0)
    t0to1 = _apply_mlp(P["p_0to1"], h0)
    t0to2 = _apply_mlp(P["p_0to2"], h0)
    t0to3 = _apply_mlp(P["p_0to3"], h0)
    t0to4 = _apply_mlp(P["p_0to4"], h0)
    t1to0 = _apply_mlp(P["p_1to0"], h1)
    t1to1 = _apply_mlp(P["p_1to1"], h1)
    t1to2 = _apply_mlp(P["p_1to2"], h1)
    t1to3 = _apply_mlp(P["p_1to3"], h1)
    t1to4 = _apply_mlp(P["p_1to4"], h1)
    t2to0 = _apply_mlp(P["p_2to0"], h2)
    t2to1 = _apply_mlp(P["p_2to1"], h2)
    t2to2 = _apply_mlp(P["p_2to2"], h2)
    t2to3 = _apply_mlp(P["p_2to3"], h2)
    t2to4 = _apply_mlp(P["p_2to4"], h2)
    t3to0 = _apply_mlp(P["p_3to0"], h3_plus) + _apply_mlp(P["p_3to0"], h3_minus)
    t3to1 = _apply_mlp(P["p_3to1"], h3_plus) + _apply_mlp(P["p_3to1"], h3_minus)
    t3to2 = _apply_mlp(P["p_3to2"], h3_plus) + _apply_mlp(P["p_3to2"], h3_minus)
    t4to0 = _apply_mlp(P["p_4to0"], h4)
    t4to1 = _apply_mlp(P["p_4to1"], h4)
    t4to2 = _apply_mlp(P["p_4to2"], h4)

    m0to0 = pr("a010", t0to0)
    m0to1 = pr("b01", t0to1)
    m0to4 = pr("b04", t0to4, mean=True)
    m1to0 = pr("b01", t1to0, T=True)
    m1to1 = pr("a101", t1to1)
    m1to4 = pr("b14", t1to4, mean=True)
    m4to0 = pr("b04", t4to0, T=True)
    m4to1 = pr("b14", t4to1, T=True)
    m0to2 = pr("b02", t0to2)
    m0to3 = pr("b03", t0to3)
    m1to2 = pr("b12", t1to2)
    m1to3 = pr("b13", t1to3)
    m2to0v = pr("b02", t2to0, T=True)
    m2to1v = pr("b12", t2to1, T=True)
    m2to2 = pr("a232", t2to2)
    m2to3 = pr("b23", t2to3)
    m2to4v = pr("b24", t2to4)
    m3to0 = pr("b03", t3to0, T=True)
    m3to1 = pr("b13", t3to1, T=True)
    m3to2 = pr("b23", t3to2, T=True)
    m4to2 = pr("b24", t4to2, T=True)

    def out_mlp(p, parts):
        x = jnp.concatenate(parts, axis=1)
        return _apply_mlp(p, x)

    msgs3 = (m0to3, m1to3, m2to3)
    h3p_minus = out_mlp(P["p_3"], (h3_minus,) + msgs3)
    h3p_plus = out_mlp(P["p_3"], (h3_plus,) + msgs3)
    h2p = out_mlp(P["p_2"], (h2, m0to2, m1to2, m2to2, m3to2, m4to2))
    h0p = out_mlp(P["p_0"], (h0, m0to0, m1to0, m2to0v, m3to0, m4to0))
    h1p = out_mlp(P["p_1"], (h1, m0to1, m1to1, m2to1v, m3to1, m4to1))
    h4p = out_mlp(P["p_4"], (h4, m0to4, m1to4, m2to4v))
    return (h0p, h1p, h2p, h3p_minus, h3p_plus, h4p)
